# Initial kernel scaffold; baseline (speedup 1.0000x reference)
import jax
import jax.numpy as jnp
from jax.experimental import pallas as pl


def kernel(x_author, x_paper, edge_index_writes, edge_index_cites, Wkqv_author, bkqv_author, Wkqv_paper, bkqv_paper, k_rel_w, k_rel_b, v_rel_w, v_rel_b, Wout_author, bout_author, Wout_paper, bout_paper, skip_author, skip_paper, p_rel_writes, p_rel_cites, att_author, att_paper):
    raise NotImplementedError("write your pallas kernel here")


# speedup vs baseline: 3.1493x; 3.1493x over previous
import functools
import math

import jax
import jax.numpy as jnp
from jax import lax
from jax.experimental import pallas as pl
from jax.experimental.pallas import tpu as pltpu

NA, NPAP, C, H = 10000, 10000, 128, 8
D = C // H
NUM_ET = 2
E_WRITES, E_CITES = 160000, 320000
E_TOT = E_WRITES + E_CITES


def _proj_body(xa_ref, xp_ref, wa_ref, ba_ref, wp_ref, bp_ref, kva_ref, kqvp_ref):
    kva_ref[...] = jnp.dot(xa_ref[...], wa_ref[...],
                           preferred_element_type=jnp.float32) + ba_ref[...]
    kqvp_ref[...] = jnp.dot(xp_ref[...], wp_ref[...],
                            preferred_element_type=jnp.float32) + bp_ref[...]


def _projections(x_author, x_paper, Wa, ba, Wp, bp):
    BLK = 2000
    grid = (NA // BLK,)
    return pl.pallas_call(
        _proj_body,
        grid=grid,
        in_specs=[
            pl.BlockSpec((BLK, C), lambda i: (i, 0)),
            pl.BlockSpec((BLK, C), lambda i: (i, 0)),
            pl.BlockSpec((C, 2 * C), lambda i: (0, 0)),
            pl.BlockSpec((1, 2 * C), lambda i: (0, 0)),
            pl.BlockSpec((C, 3 * C), lambda i: (0, 0)),
            pl.BlockSpec((1, 3 * C), lambda i: (0, 0)),
        ],
        out_specs=[
            pl.BlockSpec((BLK, 2 * C), lambda i: (i, 0)),
            pl.BlockSpec((BLK, 3 * C), lambda i: (i, 0)),
        ],
        out_shape=[
            jax.ShapeDtypeStruct((NA, 2 * C), jnp.float32),
            jax.ShapeDtypeStruct((NPAP, 3 * C), jnp.float32),
        ],
    )(x_author, x_paper, Wa, ba, Wp, bp)


def _out_body(aggr_ref, xp_ref, xa_ref, wout_ref, bout_ref, attp_ref, atta_ref,
              scal_ref, out_ref, outp_scr, zp_scr):
    sp = scal_ref[0, 0]
    sa = scal_ref[0, 1]
    ga_dot_atta = scal_ref[0, 2]
    t = jnp.dot(aggr_ref[...], wout_ref[...],
                preferred_element_type=jnp.float32) + bout_ref[...]
    out_p = 0.5 * t * (1.0 + lax.erf(t * 0.7071067811865476))
    out_p = sp * out_p + (1.0 - sp) * xp_ref[...]
    outp_scr[...] = out_p
    zp = jnp.dot(out_p, attp_ref[...], preferred_element_type=jnp.float32)
    zp_scr[...] = zp
    wp = jax.nn.softmax(zp, axis=0)
    pooled_p = jnp.dot(wp.T, outp_scr[...], preferred_element_type=jnp.float32)

    za = jnp.dot(xa_ref[...], atta_ref[...], preferred_element_type=jnp.float32)
    za = sa * ga_dot_atta + (1.0 - sa) * za
    wa = jax.nn.softmax(za, axis=0)
    pooled_a = jnp.dot(wa.T, xa_ref[...], preferred_element_type=jnp.float32)
    out_ref[0, :] = pooled_a[0, :] * (1.0 - sa)
    out_ref[1, :] = pooled_p[0, :]


def _output_stage(aggr_p, x_paper, x_author, Wout_p, bout_p, att_p, att_a,
                  scalars):
    return pl.pallas_call(
        _out_body,
        in_specs=[
            pl.BlockSpec((NPAP, C), lambda: (0, 0)),
            pl.BlockSpec((NPAP, C), lambda: (0, 0)),
            pl.BlockSpec((NA, C), lambda: (0, 0)),
            pl.BlockSpec((C, C), lambda: (0, 0)),
            pl.BlockSpec((1, C), lambda: (0, 0)),
            pl.BlockSpec((C, 1), lambda: (0, 0)),
            pl.BlockSpec((C, 1), lambda: (0, 0)),
            pl.BlockSpec((1, 8), lambda: (0, 0)),
        ],
        out_specs=pl.BlockSpec((2, C), lambda: (0, 0)),
        out_shape=jax.ShapeDtypeStruct((2, C), jnp.float32),
        scratch_shapes=[
            pltpu.VMEM((NPAP, C), jnp.float32),
            pltpu.VMEM((NPAP, 1), jnp.float32),
        ],
    )(aggr_p, x_paper, x_author, Wout_p, bout_p, att_p, att_a, scalars)


def _fold_weights(Wkqv_a, bkqv_a, Wkqv_p, bkqv_p, k_rel_w, k_rel_b,
                  v_rel_w, v_rel_b, p_rel_writes, p_rel_cites):
    Wk_a, _, Wv_a = Wkqv_a[:, :C], Wkqv_a[:, C:2 * C], Wkqv_a[:, 2 * C:]
    bk_a, _, bv_a = bkqv_a[:C], bkqv_a[C:2 * C], bkqv_a[2 * C:]
    Wk_p, Wq_p, Wv_p = Wkqv_p[:, :C], Wkqv_p[:, C:2 * C], Wkqv_p[:, 2 * C:]
    bk_p, bq_p, bv_p = bkqv_p[:C], bkqv_p[C:2 * C], bkqv_p[2 * C:]

    kw = k_rel_w.reshape(H, NUM_ET, D, D)
    kb = k_rel_b.reshape(H, NUM_ET, D)
    vw = v_rel_w.reshape(H, NUM_ET, D, D)
    vb = v_rel_b.reshape(H, NUM_ET, D)

    s_a = (p_rel_writes.reshape(H) / math.sqrt(D))[:, None]
    s_p = (p_rel_cites.reshape(H) / math.sqrt(D))[:, None]

    def fold(Wpart, bpart, rw, rb, scale):
        Wf = jnp.einsum('chd,hde->che', Wpart.reshape(C, H, D), rw)
        bf = jnp.einsum('hd,hde->he', bpart.reshape(H, D), rw) + rb
        Wf = (Wf * scale[None, :, :]).reshape(C, C)
        bf = (bf * scale).reshape(C)
        return Wf, bf

    one = jnp.ones((H, 1), jnp.float32)
    Wk_af, bk_af = fold(Wk_a, bk_a, kw[:, 0], kb[:, 0], s_a)
    Wv_af, bv_af = fold(Wv_a, bv_a, vw[:, 0], vb[:, 0], one)
    Wk_pf, bk_pf = fold(Wk_p, bk_p, kw[:, 1], kb[:, 1], s_p)
    Wv_pf, bv_pf = fold(Wv_p, bv_p, vw[:, 1], vb[:, 1], one)

    Wa = jnp.concatenate([Wk_af, Wv_af], axis=1)
    ba = jnp.concatenate([bk_af, bv_af])[None, :]
    Wp = jnp.concatenate([Wk_pf, Wq_p, Wv_pf], axis=1)
    bp = jnp.concatenate([bk_pf, bq_p, bv_pf])[None, :]
    return Wa, ba, Wp, bp


def _edge_phase(kv_a, kqv_p, ei_writes, ei_cites):
    k_a = kv_a[:, :C]
    v_a = kv_a[:, C:]
    k_p = kqv_p[:, :C]
    q_p = kqv_p[:, C:2 * C]
    v_p = kqv_p[:, 2 * C:]
    k_all = jnp.concatenate([k_a, k_p], axis=0)
    v_all = jnp.concatenate([v_a, v_p], axis=0)
    src = jnp.concatenate([ei_writes[0], ei_cites[0] + NA]).astype(jnp.int32)
    dst = jnp.concatenate([ei_writes[1], ei_cites[1]]).astype(jnp.int32)
    qe = q_p[dst].reshape(-1, H, D)
    ke = k_all[src].reshape(-1, H, D)
    ex = jnp.exp(jnp.sum(qe * ke, axis=-1))
    denom = jax.ops.segment_sum(ex, dst, num_segments=NPAP)
    num = jax.ops.segment_sum(
        (v_all[src].reshape(-1, H, D) * ex[:, :, None]).reshape(-1, C),
        dst, num_segments=NPAP)
    return num / (jnp.repeat(denom, D, axis=1) + 1e-16)


def kernel(x_author, x_paper, edge_index_writes, edge_index_cites,
           Wkqv_author, bkqv_author, Wkqv_paper, bkqv_paper,
           k_rel_w, k_rel_b, v_rel_w, v_rel_b,
           Wout_author, bout_author, Wout_paper, bout_paper,
           skip_author, skip_paper, p_rel_writes, p_rel_cites,
           att_author, att_paper):
    Wa, ba, Wp, bp = _fold_weights(Wkqv_author, bkqv_author, Wkqv_paper,
                                   bkqv_paper, k_rel_w, k_rel_b, v_rel_w,
                                   v_rel_b, p_rel_writes, p_rel_cites)
    kv_a, kqv_p = _projections(x_author, x_paper, Wa, ba, Wp, bp)

    aggr_p = _edge_phase(kv_a, kqv_p, edge_index_writes, edge_index_cites)

    sa = jax.nn.sigmoid(skip_author[0])
    sp = jax.nn.sigmoid(skip_paper[0])
    g_a = jax.nn.gelu(bout_author, approximate=False)
    ga_dot_atta = jnp.dot(g_a, att_author)
    scalars = jnp.zeros((1, 8), jnp.float32)
    scalars = scalars.at[0, 0].set(sp).at[0, 1].set(sa).at[0, 2].set(ga_dot_atta)

    pooled = _output_stage(aggr_p, x_paper, x_author, Wout_paper,
                           bout_paper[None, :], att_paper[:, None],
                           att_author[:, None], scalars)
    pooled_a = pooled[0] + sa * g_a
    pooled_p = pooled[1]
    return jnp.concatenate([pooled_a, pooled_p])[None, :]


# speedup vs baseline: 3.2388x; 1.0284x over previous
import math

import jax
import jax.numpy as jnp
from jax import lax
from jax.experimental import pallas as pl
from jax.experimental.pallas import tpu as pltpu

NA, NPAP, C, H = 10000, 10000, 128, 8
D = C // H
NUM_ET = 2
E_WRITES, E_CITES = 160000, 320000


def _proj_body(xa_ref, xp_ref, wa_ref, ba_ref, wp_ref, bp_ref,
               kva_ref, kvp_ref, qp_ref):
    kva_ref[...] = jnp.dot(xa_ref[...], wa_ref[...],
                           preferred_element_type=jnp.float32) + ba_ref[...]
    t = jnp.dot(xp_ref[...], wp_ref[...],
                preferred_element_type=jnp.float32) + bp_ref[...]
    kvp_ref[...] = t[:, :2 * C]
    qp_ref[...] = t[:, 2 * C:]


def _projections(x_author, x_paper, Wa, ba, Wp, bp):
    BLK = 2000
    grid = (NA // BLK,)
    return pl.pallas_call(
        _proj_body,
        grid=grid,
        in_specs=[
            pl.BlockSpec((BLK, C), lambda i: (i, 0)),
            pl.BlockSpec((BLK, C), lambda i: (i, 0)),
            pl.BlockSpec((C, 2 * C), lambda i: (0, 0)),
            pl.BlockSpec((1, 2 * C), lambda i: (0, 0)),
            pl.BlockSpec((C, 3 * C), lambda i: (0, 0)),
            pl.BlockSpec((1, 3 * C), lambda i: (0, 0)),
        ],
        out_specs=[
            pl.BlockSpec((BLK, 2 * C), lambda i: (i, 0)),
            pl.BlockSpec((BLK, 2 * C), lambda i: (i, 0)),
            pl.BlockSpec((BLK, C), lambda i: (i, 0)),
        ],
        out_shape=[
            jax.ShapeDtypeStruct((NA, 2 * C), jnp.float32),
            jax.ShapeDtypeStruct((NPAP, 2 * C), jnp.float32),
            jax.ShapeDtypeStruct((NPAP, C), jnp.float32),
        ],
    )(x_author, x_paper, Wa, ba, Wp, bp)


def _out_body(num_ref, den_ref, xp_ref, xa_ref, wout_ref, bout_ref,
              attp_ref, atta_ref, scal_ref, out_ref, outp_scr, zp_scr):
    sp = scal_ref[0, 0]
    sa = scal_ref[0, 1]
    ga_dot_atta = scal_ref[0, 2]
    sel = (lax.broadcasted_iota(jnp.int32, (H, C), 1) // D ==
           lax.broadcasted_iota(jnp.int32, (H, C), 0)).astype(jnp.float32)
    den_rep = jnp.dot(den_ref[...], sel, preferred_element_type=jnp.float32)
    aggr = num_ref[...] / (den_rep + 1e-16)
    t = jnp.dot(aggr, wout_ref[...],
                preferred_element_type=jnp.float32) + bout_ref[...]
    out_p = 0.5 * t * (1.0 + lax.erf(t * 0.7071067811865476))
    out_p = sp * out_p + (1.0 - sp) * xp_ref[...]
    outp_scr[...] = out_p
    zp = jnp.dot(out_p, attp_ref[...], preferred_element_type=jnp.float32)
    zp_scr[...] = zp
    wp = jax.nn.softmax(zp, axis=0)
    pooled_p = jnp.dot(wp.T, outp_scr[...], preferred_element_type=jnp.float32)

    za = jnp.dot(xa_ref[...], atta_ref[...], preferred_element_type=jnp.float32)
    za = sa * ga_dot_atta + (1.0 - sa) * za
    wa = jax.nn.softmax(za, axis=0)
    pooled_a = jnp.dot(wa.T, xa_ref[...], preferred_element_type=jnp.float32)
    out_ref[0, :] = pooled_a[0, :] * (1.0 - sa)
    out_ref[1, :] = pooled_p[0, :]


def _output_stage(num, den, x_paper, x_author, Wout_p, bout_p,
                  att_p, att_a, scalars):
    return pl.pallas_call(
        _out_body,
        in_specs=[
            pl.BlockSpec((NPAP, C), lambda: (0, 0)),
            pl.BlockSpec((NPAP, H), lambda: (0, 0)),
            pl.BlockSpec((NPAP, C), lambda: (0, 0)),
            pl.BlockSpec((NA, C), lambda: (0, 0)),
            pl.BlockSpec((C, C), lambda: (0, 0)),
            pl.BlockSpec((1, C), lambda: (0, 0)),
            pl.BlockSpec((C, 1), lambda: (0, 0)),
            pl.BlockSpec((C, 1), lambda: (0, 0)),
            pl.BlockSpec((1, 8), lambda: (0, 0)),
        ],
        out_specs=pl.BlockSpec((2, C), lambda: (0, 0)),
        out_shape=jax.ShapeDtypeStruct((2, C), jnp.float32),
        scratch_shapes=[
            pltpu.VMEM((NPAP, C), jnp.float32),
            pltpu.VMEM((NPAP, 1), jnp.float32),
        ],
    )(num, den, x_paper, x_author, Wout_p, bout_p, att_p, att_a, scalars)


def _fold_weights(Wkqv_a, bkqv_a, Wkqv_p, bkqv_p, k_rel_w, k_rel_b,
                  v_rel_w, v_rel_b, p_rel_writes, p_rel_cites):
    Wk_a, _, Wv_a = Wkqv_a[:, :C], Wkqv_a[:, C:2 * C], Wkqv_a[:, 2 * C:]
    bk_a, _, bv_a = bkqv_a[:C], bkqv_a[C:2 * C], bkqv_a[2 * C:]
    Wk_p, Wq_p, Wv_p = Wkqv_p[:, :C], Wkqv_p[:, C:2 * C], Wkqv_p[:, 2 * C:]
    bk_p, bq_p, bv_p = bkqv_p[:C], bkqv_p[C:2 * C], bkqv_p[2 * C:]

    kw = k_rel_w.reshape(H, NUM_ET, D, D)
    kb = k_rel_b.reshape(H, NUM_ET, D)
    vw = v_rel_w.reshape(H, NUM_ET, D, D)
    vb = v_rel_b.reshape(H, NUM_ET, D)

    s_a = (p_rel_writes.reshape(H) / math.sqrt(D))[:, None]
    s_p = (p_rel_cites.reshape(H) / math.sqrt(D))[:, None]

    def fold(Wpart, bpart, rw, rb, scale):
        Wf = jnp.einsum('chd,hde->che', Wpart.reshape(C, H, D), rw)
        bf = jnp.einsum('hd,hde->he', bpart.reshape(H, D), rw) + rb
        Wf = (Wf * scale[None, :, :]).reshape(C, C)
        bf = (bf * scale).reshape(C)
        return Wf, bf

    one = jnp.ones((H, 1), jnp.float32)
    Wk_af, bk_af = fold(Wk_a, bk_a, kw[:, 0], kb[:, 0], s_a)
    Wv_af, bv_af = fold(Wv_a, bv_a, vw[:, 0], vb[:, 0], one)
    Wk_pf, bk_pf = fold(Wk_p, bk_p, kw[:, 1], kb[:, 1], s_p)
    Wv_pf, bv_pf = fold(Wv_p, bv_p, vw[:, 1], vb[:, 1], one)

    Wa = jnp.concatenate([Wk_af, Wv_af], axis=1)
    ba = jnp.concatenate([bk_af, bv_af])[None, :]
    Wp = jnp.concatenate([Wk_pf, Wv_pf, Wq_p], axis=1)
    bp = jnp.concatenate([bk_pf, bv_pf, bq_p])[None, :]
    return Wa, ba, Wp, bp


def _edge_phase(kv_a, kv_p, q_p, ei_writes, ei_cites):
    k_all = jnp.concatenate([kv_a[:, :C], kv_p[:, :C]], axis=0)
    v_all = jnp.concatenate([kv_a[:, C:], kv_p[:, C:]], axis=0)
    src = jnp.concatenate([ei_writes[0], ei_cites[0] + NA]).astype(jnp.int32)
    dst = jnp.concatenate([ei_writes[1], ei_cites[1]]).astype(jnp.int32)
    qe = q_p[dst].reshape(-1, H, D)
    ke = k_all[src].reshape(-1, H, D)
    ex = jnp.exp(jnp.sum(qe * ke, axis=-1))
    den = jax.ops.segment_sum(ex, dst, num_segments=NPAP)
    num = jax.ops.segment_sum(
        (v_all[src].reshape(-1, H, D) * ex[:, :, None]).reshape(-1, C),
        dst, num_segments=NPAP)
    return num, den


def kernel(x_author, x_paper, edge_index_writes, edge_index_cites,
           Wkqv_author, bkqv_author, Wkqv_paper, bkqv_paper,
           k_rel_w, k_rel_b, v_rel_w, v_rel_b,
           Wout_author, bout_author, Wout_paper, bout_paper,
           skip_author, skip_paper, p_rel_writes, p_rel_cites,
           att_author, att_paper):
    Wa, ba, Wp, bp = _fold_weights(Wkqv_author, bkqv_author, Wkqv_paper,
                                   bkqv_paper, k_rel_w, k_rel_b, v_rel_w,
                                   v_rel_b, p_rel_writes, p_rel_cites)
    kv_a, kv_p, q_p = _projections(x_author, x_paper, Wa, ba, Wp, bp)

    num, den = _edge_phase(kv_a, kv_p, q_p, edge_index_writes,
                           edge_index_cites)

    sa = jax.nn.sigmoid(skip_author[0])
    sp = jax.nn.sigmoid(skip_paper[0])
    g_a = jax.nn.gelu(bout_author, approximate=False)
    ga_dot_atta = jnp.dot(g_a, att_author)
    scalars = jnp.zeros((1, 8), jnp.float32)
    scalars = scalars.at[0, 0].set(sp).at[0, 1].set(sa).at[0, 2].set(ga_dot_atta)

    pooled = _output_stage(num, den, x_paper, x_author, Wout_paper,
                           bout_paper[None, :], att_paper[:, None],
                           att_author[:, None], scalars)
    pooled_a = pooled[0] + sa * g_a
    pooled_p = pooled[1]
    return jnp.concatenate([pooled_a, pooled_p])[None, :]
